# baseline (speedup 1.0000x reference)
import jax
import jax.numpy as jnp
from jax.experimental import pallas as pl
from jax.experimental.pallas import tpu as pltpu
from jax.experimental.pallas import tpu_sc as plsc

_LP = 256
_GATHER_WINDOW = 128
_T = 4096
_NBUF = 6
_C = 4
_NQ = 4


def _sc_gather(emb, idx2d):
    D = emb.shape[1]
    mesh = plsc.VectorSubcoreMesh(core_axis_name="c", subcore_axis_name="s")

    @pl.kernel(out_type=jax.ShapeDtypeStruct((_LP, D), emb.dtype), mesh=mesh)
    def gather_kernel(emb_hbm, idx_hbm, out_hbm):
        def body(i_vmem, o_vmem):
            pltpu.sync_copy(emb_hbm.at[i_vmem.at[0]], o_vmem)

        pltpu.emit_pipeline(
            body,
            grid=(_LP // _GATHER_WINDOW,),
            in_specs=[pl.BlockSpec((1, _GATHER_WINDOW), lambda i: (0, i))],
            out_specs=[pl.BlockSpec((_GATHER_WINDOW, D), lambda i: (i, 0))],
            core_axis_name="s",
            dimension_semantics=(pltpu.PARALLEL,),
        )(idx_hbm, out_hbm)

    return gather_kernel(emb, idx2d)


def _mlp_logits_lse(gathered, L, W1, b1r, W2, b2r):
    LP, D = gathered.shape
    H = W1.shape[1]
    V = W2.shape[1]
    T = _T
    nt_full = V // T
    tail = V - nt_full * T
    nt = nt_full + (1 if tail else 0)
    nt_sub = pl.cdiv(nt, _C)
    rq = H // _NQ

    def clamp_idx(c, j):
        return (0, jnp.minimum(c * nt_sub + j, nt - 1))

    def kfn(g_ref, w1_ref, b1_ref, w2_hbm, w2t_ref, b2_ref,
            out_ref, m_out, s_out,
            buf, h_ref, m_ref, s_ref, sems):
        c = pl.program_id(0)
        j = pl.program_id(1)
        g = c * nt_sub + j
        hi = jnp.minimum(nt_full, (c + 1) * nt_sub)

        def sub_dma(gg, q):
            return pltpu.make_async_copy(
                w2_hbm.at[pl.ds(q * rq, rq), pl.ds(gg * T, T)],
                buf.at[jax.lax.rem(gg, _NBUF), pl.ds(q * rq, rq)],
                sems.at[q, jax.lax.rem(gg, _NBUF)])

        def issue(gg):
            @pl.when(gg < hi)
            def _():
                for q in range(_NQ):
                    sub_dma(gg, q).start()

        @pl.when(j == 0)
        def _():
            lane = jax.lax.broadcasted_iota(jnp.int32, (1, LP), 1)
            maskr = (lane < L).astype(jnp.float32)
            embr = jnp.dot(maskr, g_ref[...],
                           preferred_element_type=jnp.float32)
            hr = jnp.dot(embr, w1_ref[...],
                         preferred_element_type=jnp.float32) + b1_ref[...]
            hr = jnp.maximum(hr, 0.0)
            h_ref[...] = jnp.transpose(hr, (1, 0))
            m_ref[...] = jnp.full((1, 1), -jnp.inf, jnp.float32)
            s_ref[...] = jnp.zeros((1, 1), jnp.float32)
            for p in range(_NBUF - 1):
                issue(c * nt_sub + p)

        issue(g + _NBUF - 1)

        @pl.when(g < nt_full)
        def _():
            for q in range(_NQ):
                sub_dma(g, q).wait()

        def compute_tile(read, is_tail):
            acc = None
            for k8 in range(H // 8):
                part = h_ref[k8 * 8:(k8 + 1) * 8, :] * read(k8)
                acc = part if acc is None else acc + part
            t = jnp.sum(acc, axis=0, keepdims=True) + b2_ref[...]
            if is_tail:
                col = g * T + jax.lax.broadcasted_iota(jnp.int32, (1, T), 1)
                t = jnp.where(col < V, t, -jnp.inf)
            out_ref[...] = t

            m_old = m_ref[...]
            tmax = jnp.max(t, axis=1, keepdims=True)
            m_new = jnp.maximum(m_old, tmax)
            s_ref[...] = (s_ref[...] * jnp.exp(m_old - m_new)
                          + jnp.sum(jnp.exp(t - m_new), axis=1,
                                    keepdims=True))
            m_ref[...] = m_new

        @pl.when(g < nt_full)
        def _():
            b = jax.lax.rem(g, _NBUF)
            compute_tile(lambda k8: buf[b, k8 * 8:(k8 + 1) * 8, :], False)

        if tail:
            @pl.when(g == nt_full)
            def _():
                compute_tile(lambda k8: w2t_ref[k8 * 8:(k8 + 1) * 8, :],
                             True)

        @pl.when(j == nt_sub - 1)
        def _():
            m_out[...] = jnp.broadcast_to(m_ref[...], (8, 128))
            s_out[...] = jnp.broadcast_to(s_ref[...], (8, 128))

    return pl.pallas_call(
        kfn,
        grid=(_C, nt_sub),
        in_specs=[
            pl.BlockSpec((LP, D), lambda c, j: (0, 0)),
            pl.BlockSpec((D, H), lambda c, j: (0, 0)),
            pl.BlockSpec((1, H), lambda c, j: (0, 0)),
            pl.BlockSpec(memory_space=pl.ANY),
            pl.BlockSpec((H, T), lambda c, j: (0, nt - 1)),
            pl.BlockSpec((1, T), clamp_idx),
        ],
        out_specs=[
            pl.BlockSpec((1, T), clamp_idx),
            pl.BlockSpec((8, 128), lambda c, j: (c, 0)),
            pl.BlockSpec((8, 128), lambda c, j: (c, 0)),
        ],
        out_shape=[
            jax.ShapeDtypeStruct((1, V), jnp.float32),
            jax.ShapeDtypeStruct((_C * 8, 128), jnp.float32),
            jax.ShapeDtypeStruct((_C * 8, 128), jnp.float32),
        ],
        scratch_shapes=[
            pltpu.VMEM((_NBUF, H, T), jnp.float32),
            pltpu.VMEM((H, 1), jnp.float32),
            pltpu.VMEM((1, 1), jnp.float32),
            pltpu.VMEM((1, 1), jnp.float32),
            pltpu.SemaphoreType.DMA((_NQ, _NBUF)),
        ],
        compiler_params=pltpu.CompilerParams(
            dimension_semantics=("parallel", "arbitrary")),
    )(gathered, W1, b1r, W2, W2, b2r)


def _subtract_lse(logits, m_part, s_part):
    V = logits.shape[1]
    T = _T
    nt = pl.cdiv(V, T)

    def kfn(l_ref, m_ref, s_ref, o_ref):
        m = m_ref[...]
        s = s_ref[...]
        mg = jnp.max(m)
        sg = jnp.sum(s * jnp.exp(m - mg)) * (1.0 / 1024.0)
        lse = mg + jnp.log(sg)
        o_ref[...] = l_ref[...] - lse

    return pl.pallas_call(
        kfn,
        grid=(nt,),
        in_specs=[
            pl.BlockSpec((1, T), lambda j: (0, j)),
            pl.BlockSpec((_C * 8, 128), lambda j: (0, 0)),
            pl.BlockSpec((_C * 8, 128), lambda j: (0, 0)),
        ],
        out_specs=pl.BlockSpec((1, T), lambda j: (0, j)),
        out_shape=jax.ShapeDtypeStruct((1, V), jnp.float32),
        input_output_aliases={0: 0},
        compiler_params=pltpu.CompilerParams(
            dimension_semantics=("parallel",)),
    )(logits, m_part, s_part)


def kernel(inputs, emb, W1, b1, W2, b2):
    L = inputs.shape[0]
    H = W1.shape[1]
    V = W2.shape[1]
    idx = jnp.zeros((_LP,), jnp.int32).at[:L].set(inputs.astype(jnp.int32))
    gathered = _sc_gather(emb, idx.reshape(1, _LP))
    logits, m_part, s_part = _mlp_logits_lse(gathered, L, W1,
                                             b1.reshape(1, H),
                                             W2, b2.reshape(1, V))
    return _subtract_lse(logits, m_part, s_part)
